# baseline (speedup 1.0000x reference)
import functools

import jax
import jax.numpy as jnp
from jax import lax
from jax.experimental import pallas as pl
from jax.experimental.pallas import tpu as pltpu
from jax.experimental.pallas import tpu_sc as plsc

D_IN = 2048
N_LAT = 16384
TOPK = 64
NB = 4096

CHUNK = 128
NCHUNK = N_LAT // CHUNK

NEG_INF = float("-inf")


def _enc_body(x_ref, w_ref, b_ref, o_ref, m_ref):
    bm, bn = o_ref.shape
    p = (
        jnp.dot(x_ref[...], w_ref[...], preferred_element_type=jnp.float32)
        + b_ref[...]
    )
    o_ref[...] = p
    m_ref[...] = jnp.max(p.reshape(bm, bn // CHUNK, CHUNK), axis=2)[None]


def _encode(x, w_enc, b_enc2d):
    bm, bn = 256, 2048
    grid = (NB // bm, N_LAT // bn)
    return pl.pallas_call(
        _enc_body,
        grid=grid,
        in_specs=[
            pl.BlockSpec((bm, D_IN), lambda i, j: (i, 0)),
            pl.BlockSpec((D_IN, bn), lambda i, j: (0, j)),
            pl.BlockSpec((1, bn), lambda i, j: (0, j)),
        ],
        out_specs=[
            pl.BlockSpec((bm, bn), lambda i, j: (i, j)),
            pl.BlockSpec((1, bm, bn // CHUNK), lambda i, j: (j, i, 0)),
        ],
        out_shape=[
            jax.ShapeDtypeStruct((NB, N_LAT), jnp.float32),
            jax.ShapeDtypeStruct((N_LAT // bn, NB, bn // CHUNK), jnp.float32),
        ],
        compiler_params=pltpu.CompilerParams(
            dimension_semantics=("parallel", "parallel"),
        ),
    )(x, w_enc, b_enc2d)


def _merge16(ak, ai, bk, bi):
    rbk = lax.rev(bk, (0,))
    rbi = lax.rev(bi, (0,))
    sel = ak >= rbk
    hk = jnp.where(sel, ak, rbk)
    hi_ = jnp.where(sel, ai, rbi)
    lk = jnp.where(sel, rbk, ak)
    li = jnp.where(sel, rbi, ai)
    hk, hi_ = plsc.sort_key_val(hk, hi_, descending=True)
    lk, li = plsc.sort_key_val(lk, li, descending=True)
    return hk, hi_, lk, li


def _insert16(best, nk, ni):
    out = []
    for q in range(4):
        bk, bi = best[q]
        hk, hi_, nk, ni = _merge16(bk, bi, nk, ni)
        out.append((hk, hi_))
    return out


def _sc_select(pre, m):
    info = plsc.get_sparse_core_info()
    nw = info.num_cores * info.num_subcores
    rows_per_w = NB // nw
    mesh = plsc.VectorSubcoreMesh(core_axis_name="c", subcore_axis_name="s")

    @functools.partial(
        pl.kernel,
        mesh=mesh,
        out_type=[
            jax.ShapeDtypeStruct((NB, TOPK), jnp.int32),
            jax.ShapeDtypeStruct((NB, TOPK), jnp.float32),
            jax.ShapeDtypeStruct((NB, N_LAT), jnp.float32),
        ],
        scratch_types=[
            pltpu.VMEM((N_LAT,), jnp.float32),
            pltpu.VMEM((N_LAT,), jnp.float32),
            pltpu.VMEM((rows_per_w, NCHUNK), jnp.float32),
            pltpu.VMEM((N_LAT + 16,), jnp.float32),
            pltpu.VMEM((N_LAT + 16,), jnp.int32),
            pltpu.VMEM((rows_per_w, TOPK), jnp.int32),
            pltpu.VMEM((rows_per_w, TOPK), jnp.float32),
            pltpu.SemaphoreType.DMA,
            pltpu.SemaphoreType.DMA,
        ],
        compiler_params=pltpu.CompilerParams(needs_layout_passes=False),
    )
    def body(
        pre_hbm, m_hbm, idx_hbm, val_hbm, acts_hbm,
        rowa, rowb, mblk, cv, ci, oi, ov, sema, semb,
    ):
        wid = lax.axis_index("s") * info.num_cores + lax.axis_index("c")
        base = wid * rows_per_w
        last = base + rows_per_w - 1
        lane = lax.iota(jnp.int32, 16)
        neg = jnp.full((16,), NEG_INF, jnp.float32)
        zero_i = jnp.zeros((16,), jnp.int32)

        pltpu.sync_copy(m_hbm.at[pl.ds(base, rows_per_w)], mblk)
        pltpu.async_copy(pre_hbm.at[base], rowa, sema)

        def process(r, buf):
            row = base + r
            best = [(neg, zero_i)] * 4
            for k in range(NCHUNK // 16):
                kv = mblk[r, pl.ds(k * 16, 16)]
                kk, ii = plsc.sort_key_val(kv, zero_i, descending=True)
                best = _insert16(best, kk, ii)
            thr_s = jnp.min(best[3][0])
            thrv = jnp.full((16,), thr_s, jnp.float32)

            def scan_step(g, pos):
                vs, msks, cnts = [], [], []
                for u in range(16):
                    v = buf[pl.ds((g * 16 + u) * 16, 16)]
                    msk = v >= thrv
                    vs.append(v)
                    msks.append(msk)
                    cnts.append(jnp.sum(jnp.where(msk, 1, 0)))
                for u in range(16):
                    iv = (g * 16 + u) * 16 + lane
                    plsc.store_compressed(cv.at[pl.ds(pos, 16)], vs[u], mask=msks[u])
                    plsc.store_compressed(ci.at[pl.ds(pos, 16)], iv, mask=msks[u])
                    pos = pos + cnts[u]
                return pos

            cnt = lax.fori_loop(0, N_LAT // 256, scan_step, 0)
            cv[pl.ds(cnt, 16)] = neg

            def m_step(j, best):
                v = cv[pl.ds(j * 16, 16)]
                i = ci[pl.ds(j * 16, 16)]
                v, i = plsc.sort_key_val(v, i, descending=True)
                return _insert16(best, v, i)

            nvr = (cnt + 15) // 16
            best = [(neg, zero_i)] * 4
            best = lax.fori_loop(0, nvr, m_step, best)

            for q in range(4):
                ov[r, pl.ds(q * 16, 16)] = best[q][0]
                oi[r, pl.ds(q * 16, 16)] = best[q][1]

            v64v = jnp.full((16,), jnp.min(best[3][0]), jnp.float32)
            zf = jnp.zeros((16,), jnp.float32)

            def act_step(g, _):
                for u in range(8):
                    j = g * 8 + u
                    v = buf[pl.ds(j * 16, 16)]
                    buf[pl.ds(j * 16, 16)] = jnp.where(
                        v >= v64v, jnp.maximum(v, zf), zf
                    )
                return 0

            lax.fori_loop(0, N_LAT // 128, act_step, 0)
            pltpu.sync_copy(buf, acts_hbm.at[row])
            return 0

        def pair(k, _):
            r0 = base + 2 * k
            pltpu.async_copy(pre_hbm.at[r0 + 1], rowb, semb)
            pltpu.make_async_copy(pre_hbm.at[r0], rowa, sema).wait()
            process(2 * k, rowa)
            nxt = jnp.minimum(r0 + 2, last)
            pltpu.async_copy(pre_hbm.at[nxt], rowa, sema)
            pltpu.make_async_copy(pre_hbm.at[r0 + 1], rowb, semb).wait()
            process(2 * k + 1, rowb)
            return 0

        lax.fori_loop(0, rows_per_w // 2, pair, 0)
        pltpu.make_async_copy(pre_hbm.at[last], rowa, sema).wait()

        pltpu.sync_copy(oi, idx_hbm.at[pl.ds(base, rows_per_w)])
        pltpu.sync_copy(ov, val_hbm.at[pl.ds(base, rows_per_w)])

    return body(pre, m)


def _dec_body(a_ref, w_ref, b_ref, o_ref):
    k = pl.program_id(1)

    @pl.when(k == 0)
    def _():
        o_ref[...] = jnp.broadcast_to(b_ref[...], o_ref.shape)

    o_ref[...] += jnp.dot(
        a_ref[...].astype(jnp.bfloat16),
        w_ref[...],
        preferred_element_type=jnp.float32,
    )


def _decode(acts, w_dec, b_dec2d):
    bm, bk = 512, 2048
    grid = (NB // bm, N_LAT // bk)
    return pl.pallas_call(
        _dec_body,
        grid=grid,
        in_specs=[
            pl.BlockSpec((bm, bk), lambda i, k: (i, k)),
            pl.BlockSpec((bk, D_IN), lambda i, k: (k, 0)),
            pl.BlockSpec((1, D_IN), lambda i, k: (0, 0)),
        ],
        out_specs=pl.BlockSpec((bm, D_IN), lambda i, k: (i, 0)),
        out_shape=jax.ShapeDtypeStruct((NB, D_IN), jnp.float32),
        compiler_params=pltpu.CompilerParams(
            dimension_semantics=("parallel", "arbitrary"),
        ),
    )(acts, w_dec.astype(jnp.bfloat16), b_dec2d)


def kernel(x, W_enc, b_enc, W_dec, b_dec):
    pre, m3 = _encode(x, W_enc, b_enc.reshape(1, N_LAT))
    m = m3.transpose(1, 0, 2).reshape(NB, NCHUNK)
    topk_idx, topk_vals, acts = _sc_select(pre, m)
    recon = _decode(acts, W_dec, b_dec.reshape(1, D_IN))
    return (recon, acts, topk_idx)


# speedup vs baseline: 1.0591x; 1.0591x over previous
import functools

import jax
import jax.numpy as jnp
from jax import lax
from jax.experimental import pallas as pl
from jax.experimental.pallas import tpu as pltpu
from jax.experimental.pallas import tpu_sc as plsc

D_IN = 2048
N_LAT = 16384
TOPK = 64
NB = 4096

CHUNK = 128
NCHUNK = N_LAT // CHUNK

NEG_INF = float("-inf")


def _enc_body(x_ref, w_ref, b_ref, o_ref, m_ref):
    bm, bn = o_ref.shape
    p = (
        jnp.dot(x_ref[...], w_ref[...], preferred_element_type=jnp.float32)
        + b_ref[...]
    )
    o_ref[...] = p
    m_ref[...] = jnp.max(p.reshape(bm, bn // CHUNK, CHUNK), axis=2)[None]


def _encode(x, w_enc, b_enc2d):
    bm, bn = 256, 2048
    grid = (NB // bm, N_LAT // bn)
    return pl.pallas_call(
        _enc_body,
        grid=grid,
        in_specs=[
            pl.BlockSpec((bm, D_IN), lambda i, j: (i, 0)),
            pl.BlockSpec((D_IN, bn), lambda i, j: (0, j)),
            pl.BlockSpec((1, bn), lambda i, j: (0, j)),
        ],
        out_specs=[
            pl.BlockSpec((bm, bn), lambda i, j: (i, j)),
            pl.BlockSpec((1, bm, bn // CHUNK), lambda i, j: (j, i, 0)),
        ],
        out_shape=[
            jax.ShapeDtypeStruct((NB, N_LAT), jnp.float32),
            jax.ShapeDtypeStruct((N_LAT // bn, NB, bn // CHUNK), jnp.float32),
        ],
        compiler_params=pltpu.CompilerParams(
            dimension_semantics=("parallel", "parallel"),
        ),
    )(x, w_enc, b_enc2d)


def _merge16(ak, ai, bk, bi):
    rbk = lax.rev(bk, (0,))
    rbi = lax.rev(bi, (0,))
    sel = ak >= rbk
    hk = jnp.where(sel, ak, rbk)
    hi_ = jnp.where(sel, ai, rbi)
    lk = jnp.where(sel, rbk, ak)
    li = jnp.where(sel, rbi, ai)
    hk, hi_ = plsc.sort_key_val(hk, hi_, descending=True)
    lk, li = plsc.sort_key_val(lk, li, descending=True)
    return hk, hi_, lk, li


def _insert16(best, nk, ni):
    out = []
    for q in range(4):
        bk, bi = best[q]
        hk, hi_, nk, ni = _merge16(bk, bi, nk, ni)
        out.append((hk, hi_))
    return out


def _sc_select(pre, m):
    info = plsc.get_sparse_core_info()
    nw = info.num_cores * info.num_subcores
    rows_per_w = NB // nw
    mesh = plsc.VectorSubcoreMesh(core_axis_name="c", subcore_axis_name="s")

    @functools.partial(
        pl.kernel,
        mesh=mesh,
        out_type=[
            jax.ShapeDtypeStruct((NB, TOPK), jnp.int32),
            jax.ShapeDtypeStruct((NB, TOPK), jnp.float32),
        ],
        scratch_types=[
            pltpu.VMEM((N_LAT,), jnp.float32),
            pltpu.VMEM((N_LAT,), jnp.float32),
            pltpu.VMEM((rows_per_w, NCHUNK), jnp.float32),
            pltpu.VMEM((N_LAT + 16,), jnp.float32),
            pltpu.VMEM((N_LAT + 16,), jnp.int32),
            pltpu.VMEM((rows_per_w, TOPK), jnp.int32),
            pltpu.VMEM((rows_per_w, TOPK), jnp.float32),
            pltpu.SemaphoreType.DMA,
            pltpu.SemaphoreType.DMA,
        ],
        compiler_params=pltpu.CompilerParams(needs_layout_passes=False),
    )
    def body(
        pre_hbm, m_hbm, idx_hbm, val_hbm,
        rowa, rowb, mblk, cv, ci, oi, ov, sema, semb,
    ):
        wid = lax.axis_index("s") * info.num_cores + lax.axis_index("c")
        base = wid * rows_per_w
        last = base + rows_per_w - 1
        lane = lax.iota(jnp.int32, 16)
        neg = jnp.full((16,), NEG_INF, jnp.float32)
        zero_i = jnp.zeros((16,), jnp.int32)

        pltpu.sync_copy(m_hbm.at[pl.ds(base, rows_per_w)], mblk)
        pltpu.async_copy(pre_hbm.at[base], rowa, sema)

        def process(r, buf):
            row = base + r
            best = [(neg, zero_i)] * 4
            for k in range(NCHUNK // 16):
                kv = mblk[r, pl.ds(k * 16, 16)]
                kk, ii = plsc.sort_key_val(kv, zero_i, descending=True)
                best = _insert16(best, kk, ii)
            thr_s = jnp.min(best[3][0])
            thrv = jnp.full((16,), thr_s, jnp.float32)

            def scan_step(g, pos):
                vs, msks, cnts = [], [], []
                for u in range(16):
                    v = buf[pl.ds((g * 16 + u) * 16, 16)]
                    msk = v >= thrv
                    vs.append(v)
                    msks.append(msk)
                    cnts.append(jnp.sum(jnp.where(msk, 1, 0)))
                for u in range(16):
                    iv = (g * 16 + u) * 16 + lane
                    plsc.store_compressed(cv.at[pl.ds(pos, 16)], vs[u], mask=msks[u])
                    plsc.store_compressed(ci.at[pl.ds(pos, 16)], iv, mask=msks[u])
                    pos = pos + cnts[u]
                return pos

            cnt = lax.fori_loop(0, N_LAT // 256, scan_step, 0)
            cv[pl.ds(cnt, 16)] = neg

            def m_step(j, best):
                v = cv[pl.ds(j * 16, 16)]
                i = ci[pl.ds(j * 16, 16)]
                v, i = plsc.sort_key_val(v, i, descending=True)
                return _insert16(best, v, i)

            nvr = (cnt + 15) // 16
            best = [(neg, zero_i)] * 4
            best = lax.fori_loop(0, nvr, m_step, best)

            for q in range(4):
                ov[r, pl.ds(q * 16, 16)] = best[q][0]
                oi[r, pl.ds(q * 16, 16)] = best[q][1]
            return 0

        def pair(k, _):
            r0 = base + 2 * k
            pltpu.async_copy(pre_hbm.at[r0 + 1], rowb, semb)
            pltpu.make_async_copy(pre_hbm.at[r0], rowa, sema).wait()
            process(2 * k, rowa)
            nxt = jnp.minimum(r0 + 2, last)
            pltpu.async_copy(pre_hbm.at[nxt], rowa, sema)
            pltpu.make_async_copy(pre_hbm.at[r0 + 1], rowb, semb).wait()
            process(2 * k + 1, rowb)
            return 0

        lax.fori_loop(0, rows_per_w // 2, pair, 0)
        pltpu.make_async_copy(pre_hbm.at[last], rowa, sema).wait()

        pltpu.sync_copy(oi, idx_hbm.at[pl.ds(base, rows_per_w)])
        pltpu.sync_copy(ov, val_hbm.at[pl.ds(base, rows_per_w)])

    return body(pre, m)


def _dec_body(p_ref, v_ref, w_ref, b_ref, o_ref, a_ref):
    k = pl.program_id(1)
    v64 = v_ref[:, TOPK - 1 :]
    p = p_ref[...]
    a = jnp.where(p >= v64, jnp.maximum(p, 0.0), 0.0)
    a_ref[...] = a

    @pl.when(k == 0)
    def _():
        o_ref[...] = jnp.broadcast_to(b_ref[...], o_ref.shape)

    o_ref[...] += jnp.dot(
        a.astype(jnp.bfloat16), w_ref[...], preferred_element_type=jnp.float32
    )


def _decode(pre, vals, w_dec, b_dec2d):
    bm, bk = 512, 2048
    grid = (NB // bm, N_LAT // bk)
    return pl.pallas_call(
        _dec_body,
        grid=grid,
        in_specs=[
            pl.BlockSpec((bm, bk), lambda i, k: (i, k)),
            pl.BlockSpec((bm, TOPK), lambda i, k: (i, 0)),
            pl.BlockSpec((bk, D_IN), lambda i, k: (k, 0)),
            pl.BlockSpec((1, D_IN), lambda i, k: (0, 0)),
        ],
        out_specs=[
            pl.BlockSpec((bm, D_IN), lambda i, k: (i, 0)),
            pl.BlockSpec((bm, bk), lambda i, k: (i, k)),
        ],
        out_shape=[
            jax.ShapeDtypeStruct((NB, D_IN), jnp.float32),
            jax.ShapeDtypeStruct((NB, N_LAT), jnp.float32),
        ],
        compiler_params=pltpu.CompilerParams(
            dimension_semantics=("parallel", "arbitrary"),
        ),
    )(pre, vals, w_dec.astype(jnp.bfloat16), b_dec2d)


def kernel(x, W_enc, b_enc, W_dec, b_dec):
    pre, m3 = _encode(x, W_enc, b_enc.reshape(1, N_LAT))
    m = m3.transpose(1, 0, 2).reshape(NB, NCHUNK)
    topk_idx, topk_vals = _sc_select(pre, m)
    recon, acts = _decode(pre, topk_vals, W_dec, b_dec.reshape(1, D_IN))
    return (recon, acts, topk_idx)
